# Initial kernel scaffold; baseline (speedup 1.0000x reference)
import jax
import jax.numpy as jnp
from jax.experimental import pallas as pl


def kernel(atom_features, deg_slice, membership, deg_adj_1, deg_adj_2, deg_adj_3, deg_adj_4, deg_adj_5, deg_adj_6, deg_adj_7, deg_adj_8, deg_adj_9, deg_adj_10, W, b):
    raise NotImplementedError("write your pallas kernel here")


# speedup vs baseline: 3.2379x; 3.2379x over previous
import functools

import jax
import jax.numpy as jnp
from jax import lax
from jax.experimental import pallas as pl
from jax.experimental.pallas import tpu as pltpu
from jax.experimental.pallas import tpu_sc as plsc

MAX_DEG = 10
N_PER = 10000
D = 128
N = 11 * N_PER
NUM_ADJ_ROWS = MAX_DEG * (MAX_DEG + 1) // 2

NUM_WORKERS = 32
SPAN = 328
STRIDE = 312


def _sc_gather_sums(atom_features, adj_blocks):
    mesh = plsc.VectorSubcoreMesh(core_axis_name="c", subcore_axis_name="s")

    @functools.partial(
        pl.kernel,
        out_type=jax.ShapeDtypeStruct((MAX_DEG * N_PER, D), jnp.float32),
        mesh=mesh,
        scratch_types=[
            pltpu.VMEM((NUM_ADJ_ROWS * SPAN,), jnp.int32),
            pltpu.VMEM((SPAN, D), jnp.float32),
            pltpu.SemaphoreType.DMA,
        ],
    )
    def body(feats_hbm, adj_hbm, out_hbm, idx_v, acc_v, sem):
        wid = lax.axis_index("s") * 2 + lax.axis_index("c")
        base = wid * STRIDE
        pltpu.sync_copy(adj_hbm.at[wid], idx_v)
        for d in range(1, MAX_DEG + 1):
            off = d * (d - 1) // 2
            pltpu.async_copy(
                feats_hbm.at[idx_v.at[pl.ds(off * SPAN, SPAN)]], acc_v, sem
            ).wait()
            for j in range(1, d):
                pltpu.async_copy(
                    feats_hbm.at[idx_v.at[pl.ds((off + j) * SPAN, SPAN)]],
                    acc_v, sem, add=True,
                ).wait()
            pltpu.sync_copy(
                acc_v, out_hbm.at[pl.ds((d - 1) * N_PER + base, SPAN)]
            )

    return body(atom_features, adj_blocks)


def _tc_combine(atom_features, rel_sums, W, b):
    b3 = b.reshape(b.shape[0], 1, D)
    BR = 1000
    RB = N_PER // BR

    def self_w_idx(d):
        return jnp.where(d == 0, 2 * MAX_DEG, 2 * d - 1)

    def rel_w_idx(d):
        return jnp.where(d == 0, 0, 2 * d - 2)

    def body(feat_ref, sums_ref, wself_ref, wrel_ref, bself_ref, brel_ref,
             out_ref):
        d = pl.program_id(0)
        acc = (
            jnp.dot(feat_ref[...], wself_ref[0],
                    preferred_element_type=jnp.float32)
            + bself_ref[0]
        )

        @pl.when(d > 0)
        def _():
            inv = 1.0 / d.astype(jnp.float32)
            out_ref[...] = (
                acc
                + jnp.dot(sums_ref[...], wrel_ref[0],
                          preferred_element_type=jnp.float32) * inv
                + brel_ref[0]
            )

        @pl.when(d == 0)
        def _():
            out_ref[...] = acc

    return pl.pallas_call(
        body,
        grid=(MAX_DEG + 1, RB),
        in_specs=[
            pl.BlockSpec((BR, D), lambda d, r: (d * RB + r, 0)),
            pl.BlockSpec((BR, D),
                         lambda d, r: (jnp.maximum(d - 1, 0) * RB + r, 0)),
            pl.BlockSpec((1, D, D), lambda d, r: (self_w_idx(d), 0, 0)),
            pl.BlockSpec((1, D, D), lambda d, r: (rel_w_idx(d), 0, 0)),
            pl.BlockSpec((1, 1, D), lambda d, r: (self_w_idx(d), 0, 0)),
            pl.BlockSpec((1, 1, D), lambda d, r: (rel_w_idx(d), 0, 0)),
        ],
        out_specs=pl.BlockSpec((BR, D), lambda d, r: (d * RB + r, 0)),
        out_shape=jax.ShapeDtypeStruct((N, D), jnp.float32),
    )(atom_features, rel_sums, W, W, b3, b3)


def kernel(atom_features, deg_slice, membership, deg_adj_1, deg_adj_2,
           deg_adj_3, deg_adj_4, deg_adj_5, deg_adj_6, deg_adj_7, deg_adj_8,
           deg_adj_9, deg_adj_10, W, b):
    adjs = [deg_adj_1, deg_adj_2, deg_adj_3, deg_adj_4, deg_adj_5, deg_adj_6,
            deg_adj_7, deg_adj_8, deg_adj_9, deg_adj_10]
    adj_all = jnp.concatenate([a.T for a in adjs], axis=0)
    adj_blocks = jnp.stack(
        [lax.slice_in_dim(adj_all, w * STRIDE, w * STRIDE + SPAN, axis=1)
         for w in range(NUM_WORKERS)]
    ).reshape(NUM_WORKERS, NUM_ADJ_ROWS * SPAN)
    rel_sums = _sc_gather_sums(atom_features, adj_blocks)
    return _tc_combine(atom_features, rel_sums, W, b)


# speedup vs baseline: 3.4951x; 1.0795x over previous
import functools

import jax
import jax.numpy as jnp
from jax import lax
from jax.experimental import pallas as pl
from jax.experimental.pallas import tpu as pltpu
from jax.experimental.pallas import tpu_sc as plsc

MAX_DEG = 10
N_PER = 10000
D = 128
N = 11 * N_PER
NUM_ADJ_ROWS = MAX_DEG * (MAX_DEG + 1) // 2

NUM_WORKERS = 32
SPAN = 328
STRIDE = 312


def _sc_gather_sums(atom_features, adj_blocks):
    mesh = plsc.VectorSubcoreMesh(core_axis_name="c", subcore_axis_name="s")

    @functools.partial(
        pl.kernel,
        out_type=jax.ShapeDtypeStruct((MAX_DEG * N_PER, D), jnp.float32),
        mesh=mesh,
        scratch_types=[
            pltpu.VMEM((NUM_ADJ_ROWS * SPAN,), jnp.int32),
            pltpu.VMEM((SPAN, D), jnp.float32),
            pltpu.VMEM((SPAN, D), jnp.float32),
            pltpu.SemaphoreType.DMA,
            pltpu.SemaphoreType.DMA,
            pltpu.SemaphoreType.DMA,
            pltpu.SemaphoreType.DMA,
            pltpu.SemaphoreType.DMA,
            pltpu.SemaphoreType.DMA,
        ],
    )
    def body(feats_hbm, adj_hbm, out_hbm, idx_v, acc0, acc1,
             sg0, sg1, sa0, sa1, sw0, sw1):
        wid = lax.axis_index("s") * 2 + lax.axis_index("c")
        base = wid * STRIDE
        accs, sgs, sas, sws = [acc0, acc1], [sg0, sg1], [sa0, sa1], [sw0, sw1]

        def gather0(d, p):
            off = d * (d - 1) // 2
            return pltpu.async_copy(
                feats_hbm.at[idx_v.at[pl.ds(off * SPAN, SPAN)]],
                accs[p], sgs[p])

        pltpu.sync_copy(adj_hbm.at[wid], idx_v)
        g0 = {1: gather0(1, 0)}
        writes = {}
        for d in range(1, MAX_DEG + 1):
            p = (d - 1) % 2
            q = d % 2
            off = d * (d - 1) // 2
            g0[d].wait()
            if d < MAX_DEG:
                if d >= 2:
                    writes[d - 1].wait()
                g0[d + 1] = gather0(d + 1, q)
            adds = [
                pltpu.async_copy(
                    feats_hbm.at[idx_v.at[pl.ds((off + j) * SPAN, SPAN)]],
                    accs[p], sas[p], add=True)
                for j in range(1, d)
            ]
            for a in adds:
                a.wait()
            writes[d] = pltpu.async_copy(
                accs[p], out_hbm.at[pl.ds((d - 1) * N_PER + base, SPAN)],
                sws[p])
        writes[MAX_DEG - 1].wait()
        writes[MAX_DEG].wait()

    return body(atom_features, adj_blocks)


def _tc_combine(atom_features, rel_sums, W, b):
    b3 = b.reshape(b.shape[0], 1, D)
    BR = 1000
    RB = N_PER // BR

    def self_w_idx(d):
        return jnp.where(d == 0, 2 * MAX_DEG, 2 * d - 1)

    def rel_w_idx(d):
        return jnp.where(d == 0, 0, 2 * d - 2)

    def body(feat_ref, sums_ref, wself_ref, wrel_ref, bself_ref, brel_ref,
             out_ref):
        d = pl.program_id(0)
        acc = (
            jnp.dot(feat_ref[...], wself_ref[0],
                    preferred_element_type=jnp.float32)
            + bself_ref[0]
        )

        @pl.when(d > 0)
        def _():
            inv = 1.0 / d.astype(jnp.float32)
            out_ref[...] = (
                acc
                + jnp.dot(sums_ref[...], wrel_ref[0],
                          preferred_element_type=jnp.float32) * inv
                + brel_ref[0]
            )

        @pl.when(d == 0)
        def _():
            out_ref[...] = acc

    return pl.pallas_call(
        body,
        grid=(MAX_DEG + 1, RB),
        in_specs=[
            pl.BlockSpec((BR, D), lambda d, r: (d * RB + r, 0)),
            pl.BlockSpec((BR, D),
                         lambda d, r: (jnp.maximum(d - 1, 0) * RB + r, 0)),
            pl.BlockSpec((1, D, D), lambda d, r: (self_w_idx(d), 0, 0)),
            pl.BlockSpec((1, D, D), lambda d, r: (rel_w_idx(d), 0, 0)),
            pl.BlockSpec((1, 1, D), lambda d, r: (self_w_idx(d), 0, 0)),
            pl.BlockSpec((1, 1, D), lambda d, r: (rel_w_idx(d), 0, 0)),
        ],
        out_specs=pl.BlockSpec((BR, D), lambda d, r: (d * RB + r, 0)),
        out_shape=jax.ShapeDtypeStruct((N, D), jnp.float32),
    )(atom_features, rel_sums, W, W, b3, b3)


def kernel(atom_features, deg_slice, membership, deg_adj_1, deg_adj_2,
           deg_adj_3, deg_adj_4, deg_adj_5, deg_adj_6, deg_adj_7, deg_adj_8,
           deg_adj_9, deg_adj_10, W, b):
    adjs = [deg_adj_1, deg_adj_2, deg_adj_3, deg_adj_4, deg_adj_5, deg_adj_6,
            deg_adj_7, deg_adj_8, deg_adj_9, deg_adj_10]
    adj_all = jnp.concatenate([a.T for a in adjs], axis=0)
    adj_blocks = jnp.stack(
        [lax.slice_in_dim(adj_all, w * STRIDE, w * STRIDE + SPAN, axis=1)
         for w in range(NUM_WORKERS)]
    ).reshape(NUM_WORKERS, NUM_ADJ_ROWS * SPAN)
    rel_sums = _sc_gather_sums(atom_features, adj_blocks)
    return _tc_combine(atom_features, rel_sums, W, b)
